# baseline (speedup 1.0000x reference)
import jax
import jax.numpy as jnp
from jax.experimental import pallas as pl
from jax.experimental.pallas import tpu as pltpu

_B, _C, _H, _W = 1024, 32, 16, 16
_E, _K = 64, 2
_D = _C * _H * _W
_HW = _H * _W
_BB = 64


def _moe_kern(xm_ref, w9_ref, wg3_ref, bias_ref, out_ref, loss_ref,
              imp_ref, load_ref):
    i = pl.program_id(0)
    nb = pl.num_programs(0)

    @pl.when(i == 0)
    def _init():
        imp_ref[...] = jnp.zeros_like(imp_ref)
        load_ref[...] = jnp.zeros_like(load_ref)

    xm = xm_ref[...]
    x3 = xm.reshape(_BB, _C, _HW)

    lane = jax.lax.broadcasted_iota(jnp.int32, (1, 1, _HW), 2)
    wl = lane % _W
    hl = lane // _W
    taps = []
    for dh in (-1, 0, 1):
        for dw in (-1, 0, 1):
            off = dh * _W + dw
            s = jnp.roll(x3, shift=-off, axis=2) if off else x3
            ok = ((wl + dw >= 0) & (wl + dw < _W)
                  & (hl + dh >= 0) & (hl + dh < _H))
            taps.append(jnp.where(ok, s, 0.0).astype(jnp.bfloat16))
    patch = jnp.concatenate(taps, axis=1)

    w9b = jnp.broadcast_to(w9_ref[...], (_BB, _C, 9 * _C))
    acc = jax.lax.dot_general(
        w9b, patch, (((2,), (1,)), ((0,), (0,))),
        preferred_element_type=jnp.float32)
    acc += bias_ref[...][None]
    out_ref[...] = acc.reshape(_BB * _C, _HW)

    xf = (xm + jnp.float32(0.0)).reshape(_BB, _D)
    logits = jnp.dot(xf, wg3_ref[...],
                     preferred_element_type=jnp.float32)

    eiota = jax.lax.broadcasted_iota(jnp.int32, (_BB, _E), 1)
    m1 = jnp.max(logits, axis=1, keepdims=True)
    idx1 = jnp.min(jnp.where(logits == m1, eiota, _E), axis=1, keepdims=True)
    lm = jnp.where(eiota == idx1, jnp.float32(-1e30), logits)
    m2 = jnp.max(lm, axis=1, keepdims=True)
    idx2 = jnp.min(jnp.where(lm == m2, eiota, _E), axis=1, keepdims=True)
    e2 = jnp.exp(m2 - m1)
    g1 = 1.0 / (1.0 + e2)
    g2 = e2 / (1.0 + e2)

    oh1 = eiota == idx1
    oh2 = eiota == idx2
    zero = jnp.float32(0.0)
    imp_part = jnp.sum(jnp.where(oh1, g1, zero) + jnp.where(oh2, g2, zero),
                       axis=0, keepdims=True)
    load_part = jnp.sum(jnp.where(oh1 & (g1 > 0), 1.0, zero)
                        + jnp.where(oh2 & (g2 > 0), 1.0, zero),
                        axis=0, keepdims=True)
    imp_ref[...] += imp_part
    load_ref[...] += load_part

    @pl.when(i == nb - 1)
    def _fin():
        def cv2(v):
            m = jnp.sum(v, axis=(0, 1), keepdims=True) / _E
            var = jnp.sum((v - m) ** 2, axis=(0, 1), keepdims=True) / (_E - 1)
            return var / (m * m + 1e-10)

        loss_ref[...] = (cv2(imp_ref[...]) + cv2(load_ref[...])) * 1e-2


def kernel(x, w_gate, w_conv, b_conv):
    xm = x.reshape(_B * _C, _HW)
    w9 = w_conv.transpose(0, 2, 3, 1).reshape(_C, 9 * _C).astype(jnp.bfloat16)
    wg3 = w_gate
    bias = jnp.broadcast_to(b_conv[:, None], (_C, _HW))

    out_conv, loss = pl.pallas_call(
        _moe_kern,
        grid=(_B // _BB,),
        in_specs=[
            pl.BlockSpec((_BB * _C, _HW), lambda i: (i, 0)),
            pl.BlockSpec((_C, 9 * _C), lambda i: (0, 0)),
            pl.BlockSpec((_D, _E), lambda i: (0, 0)),
            pl.BlockSpec((_C, _HW), lambda i: (0, 0)),
        ],
        out_specs=[
            pl.BlockSpec((_BB * _C, _HW), lambda i: (i, 0)),
            pl.BlockSpec((1, 1), lambda i: (0, 0)),
        ],
        out_shape=[
            jax.ShapeDtypeStruct((_B * _C, _HW), jnp.float32),
            jax.ShapeDtypeStruct((1, 1), jnp.float32),
        ],
        scratch_shapes=[
            pltpu.VMEM((1, _E), jnp.float32),
            pltpu.VMEM((1, _E), jnp.float32),
        ],
        compiler_params=pltpu.CompilerParams(
            dimension_semantics=("arbitrary",)),
    )(xm, w9, wg3, bias)

    return out_conv.reshape(_B, _D), loss[0, 0]


# speedup vs baseline: 2.2533x; 2.2533x over previous
import jax
import jax.numpy as jnp
from jax.experimental import pallas as pl
from jax.experimental.pallas import tpu as pltpu

_B, _C, _H, _W = 1024, 32, 16, 16
_E, _K = 64, 2
_D = _C * _H * _W
_HW = _H * _W
_BB = 128


def _moe_kern(xr_ref, w9_ref, wg_ref, bias_ref, out_ref, loss_ref,
              imp_ref, load_ref):
    i = pl.program_id(0)
    nb = pl.num_programs(0)

    @pl.when(i == 0)
    def _init():
        imp_ref[...] = jnp.zeros_like(imp_ref)
        load_ref[...] = jnp.zeros_like(load_ref)

    xf = xr_ref[...]
    xm = xf.reshape(_BB * _C, _HW) + jnp.float32(0.0)
    xb = xm.reshape(_BB, _C, _HW)

    lane = jax.lax.broadcasted_iota(jnp.int32, (1, 1, _HW), 2)
    wl = lane % _W
    hl = lane // _W
    taps = []
    for dh in (-1, 0, 1):
        for dw in (-1, 0, 1):
            off = dh * _W + dw
            s = jnp.roll(xb, shift=-off, axis=2) if off else xb
            ok = ((wl + dw >= 0) & (wl + dw < _W)
                  & (hl + dh >= 0) & (hl + dh < _H))
            taps.append(jnp.where(ok, s, 0.0).astype(jnp.bfloat16))
    patch = jnp.concatenate(taps, axis=1)

    w9b = jnp.broadcast_to(w9_ref[...], (_BB, _C, 9 * _C))
    acc = jax.lax.dot_general(
        w9b, patch, (((2,), (1,)), ((0,), (0,))),
        preferred_element_type=jnp.float32)
    acc += bias_ref[...][None]
    am = acc.reshape(_BB * _C, _HW) + jnp.float32(0.0)
    out_ref[...] = am.reshape(_BB, _D)

    logits = jnp.dot(xf, wg_ref[...],
                     preferred_element_type=jnp.float32)

    eiota = jax.lax.broadcasted_iota(jnp.int32, (_BB, _E), 1)
    m1 = jnp.max(logits, axis=1, keepdims=True)
    idx1 = jnp.min(jnp.where(logits == m1, eiota, _E), axis=1, keepdims=True)
    lm = jnp.where(eiota == idx1, jnp.float32(-1e30), logits)
    m2 = jnp.max(lm, axis=1, keepdims=True)
    idx2 = jnp.min(jnp.where(lm == m2, eiota, _E), axis=1, keepdims=True)
    e2 = jnp.exp(m2 - m1)
    g1 = 1.0 / (1.0 + e2)
    g2 = e2 / (1.0 + e2)

    oh1 = eiota == idx1
    oh2 = eiota == idx2
    zero = jnp.float32(0.0)
    imp_part = jnp.sum(jnp.where(oh1, g1, zero) + jnp.where(oh2, g2, zero),
                       axis=0, keepdims=True)
    load_part = jnp.sum(jnp.where(oh1 & (g1 > 0), 1.0, zero)
                        + jnp.where(oh2 & (g2 > 0), 1.0, zero),
                        axis=0, keepdims=True)
    imp_ref[...] += imp_part
    load_ref[...] += load_part

    @pl.when(i == nb - 1)
    def _fin():
        def cv2(v):
            m = jnp.sum(v, axis=(0, 1), keepdims=True) / _E
            var = jnp.sum((v - m) ** 2, axis=(0, 1), keepdims=True) / (_E - 1)
            return var / (m * m + 1e-10)

        loss_ref[...] = (cv2(imp_ref[...]) + cv2(load_ref[...])) * 1e-2


def kernel(x, w_gate, w_conv, b_conv):
    xr = x.reshape(_B, _D)
    w9 = w_conv.transpose(0, 2, 3, 1).reshape(_C, 9 * _C).astype(jnp.bfloat16)
    bias = jnp.broadcast_to(b_conv[:, None], (_C, _HW))

    out_conv, loss = pl.pallas_call(
        _moe_kern,
        grid=(_B // _BB,),
        in_specs=[
            pl.BlockSpec((_BB, _D), lambda i: (i, 0)),
            pl.BlockSpec((_C, 9 * _C), lambda i: (0, 0)),
            pl.BlockSpec((_D, _E), lambda i: (0, 0)),
            pl.BlockSpec((_C, _HW), lambda i: (0, 0)),
        ],
        out_specs=[
            pl.BlockSpec((_BB, _D), lambda i: (i, 0)),
            pl.BlockSpec((1, 1), lambda i: (0, 0)),
        ],
        out_shape=[
            jax.ShapeDtypeStruct((_B, _D), jnp.float32),
            jax.ShapeDtypeStruct((1, 1), jnp.float32),
        ],
        scratch_shapes=[
            pltpu.VMEM((1, _E), jnp.float32),
            pltpu.VMEM((1, _E), jnp.float32),
        ],
        compiler_params=pltpu.CompilerParams(
            dimension_semantics=("arbitrary",)),
    )(xr, w9, w_gate, bias)

    return out_conv, loss[0, 0]


# speedup vs baseline: 2.2826x; 1.0130x over previous
import jax
import jax.numpy as jnp
from jax.experimental import pallas as pl
from jax.experimental.pallas import tpu as pltpu

_B, _C, _H, _W = 1024, 32, 16, 16
_E, _K = 64, 2
_D = _C * _H * _W
_HW = _H * _W
_BB = 128


def _moe_kern(xr_ref, w9_ref, wg_ref, bias_ref, out_ref, loss_ref,
              imp_ref, load_ref):
    i = pl.program_id(0)
    nb = pl.num_programs(0)

    @pl.when(i == 0)
    def _init():
        imp_ref[...] = jnp.zeros_like(imp_ref)
        load_ref[...] = jnp.zeros_like(load_ref)

    xf = xr_ref[...]
    xm = xf.reshape(_BB * _C, _HW) + jnp.float32(0.0)
    xb = xm.reshape(_BB, _C, _HW)

    lane = jax.lax.broadcasted_iota(jnp.int32, (1, 1, _HW), 2)
    wl = lane % _W
    hl = lane // _W
    acc = jnp.broadcast_to(bias_ref[...][None], (_BB, _C, _HW))
    for r, dh in enumerate((-1, 0, 1)):
        taps = []
        for dw in (-1, 0, 1):
            off = dh * _W + dw
            s = jnp.roll(xb, shift=-off, axis=2) if off else xb
            ok = ((wl + dw >= 0) & (wl + dw < _W)
                  & (hl + dh >= 0) & (hl + dh < _H))
            taps.append(jnp.where(ok, s, 0.0).astype(jnp.bfloat16))
        patch = jnp.concatenate(taps, axis=1)
        w9b = jnp.broadcast_to(w9_ref[...][:, r * 3 * _C:(r + 1) * 3 * _C],
                               (_BB, _C, 3 * _C))
        acc = acc + jax.lax.dot_general(
            w9b, patch, (((2,), (1,)), ((0,), (0,))),
            preferred_element_type=jnp.float32)
    am = acc.reshape(_BB * _C, _HW) + jnp.float32(0.0)
    out_ref[...] = am.reshape(_BB, _D)

    logits = jnp.dot(xf, wg_ref[...],
                     preferred_element_type=jnp.float32)

    eiota = jax.lax.broadcasted_iota(jnp.int32, (_BB, _E), 1)
    m1 = jnp.max(logits, axis=1, keepdims=True)
    idx1 = jnp.min(jnp.where(logits == m1, eiota, _E), axis=1, keepdims=True)
    lm = jnp.where(eiota == idx1, jnp.float32(-1e30), logits)
    m2 = jnp.max(lm, axis=1, keepdims=True)
    idx2 = jnp.min(jnp.where(lm == m2, eiota, _E), axis=1, keepdims=True)
    e2 = jnp.exp(m2 - m1)
    g1 = 1.0 / (1.0 + e2)
    g2 = e2 / (1.0 + e2)

    oh1 = eiota == idx1
    oh2 = eiota == idx2
    zero = jnp.float32(0.0)
    imp_part = jnp.sum(jnp.where(oh1, g1, zero) + jnp.where(oh2, g2, zero),
                       axis=0, keepdims=True)
    load_part = jnp.sum(jnp.where(oh1 & (g1 > 0), 1.0, zero)
                        + jnp.where(oh2 & (g2 > 0), 1.0, zero),
                        axis=0, keepdims=True)
    imp_ref[...] += imp_part
    load_ref[...] += load_part

    @pl.when(i == nb - 1)
    def _fin():
        def cv2(v):
            m = jnp.sum(v, axis=(0, 1), keepdims=True) / _E
            var = jnp.sum((v - m) ** 2, axis=(0, 1), keepdims=True) / (_E - 1)
            return var / (m * m + 1e-10)

        loss_ref[...] = (cv2(imp_ref[...]) + cv2(load_ref[...])) * 1e-2


def kernel(x, w_gate, w_conv, b_conv):
    xr = x.reshape(_B, _D)
    w9 = w_conv.transpose(0, 2, 3, 1).reshape(_C, 9 * _C).astype(jnp.bfloat16)
    bias = jnp.broadcast_to(b_conv[:, None], (_C, _HW))

    out_conv, loss = pl.pallas_call(
        _moe_kern,
        grid=(_B // _BB,),
        in_specs=[
            pl.BlockSpec((_BB, _D), lambda i: (i, 0)),
            pl.BlockSpec((_C, 9 * _C), lambda i: (0, 0)),
            pl.BlockSpec((_D, _E), lambda i: (0, 0)),
            pl.BlockSpec((_C, _HW), lambda i: (0, 0)),
        ],
        out_specs=[
            pl.BlockSpec((_BB, _D), lambda i: (i, 0)),
            pl.BlockSpec((1, 1), lambda i: (0, 0)),
        ],
        out_shape=[
            jax.ShapeDtypeStruct((_B, _D), jnp.float32),
            jax.ShapeDtypeStruct((1, 1), jnp.float32),
        ],
        scratch_shapes=[
            pltpu.VMEM((1, _E), jnp.float32),
            pltpu.VMEM((1, _E), jnp.float32),
        ],
        compiler_params=pltpu.CompilerParams(
            dimension_semantics=("arbitrary",)),
    )(xr, w9, w_gate, bias)

    return out_conv, loss[0, 0]
